# Initial kernel scaffold; baseline (speedup 1.0000x reference)
import jax
import jax.numpy as jnp
from jax.experimental import pallas as pl


def kernel(Z, Y):
    raise NotImplementedError("write your pallas kernel here")


# speedup vs baseline: 3.7346x; 3.7346x over previous
import jax
import jax.numpy as jnp
from jax.experimental import pallas as pl
from jax.experimental.pallas import tpu as pltpu


def _body(y_ref, z_ref, top1_ref, top10_ref, cnt_ref, diag_ref, yn_ref,
          *, bi, bj, nj):
    i = pl.program_id(0)
    j = pl.program_id(1)

    y = y_ref[...]
    z = z_ref[...]

    @pl.when(j == 0)
    def _init_row_block():
        cnt_ref[...] = jnp.zeros_like(cnt_ref)
        yn_ref[...] = jnp.sqrt(jnp.sum(y * y, axis=1))[:, None]

    @pl.when((i == 0) & (j == 0))
    def _init_outputs():
        top1_ref[...] = jnp.zeros_like(top1_ref)
        top10_ref[...] = jnp.zeros_like(top10_ref)

    dots = jax.lax.dot_general(
        y, z, (((1,), (1,)), ((), ())),
        preferred_element_type=jnp.float32)
    xn = jnp.sqrt(jnp.sum(z * z, axis=1))
    denom = jnp.maximum(yn_ref[...] * xn[None, :], 1e-8)
    sim = dots / denom

    jd = (i * bi) // bj
    jp = jax.lax.rem(jd + j, nj)
    row_g = i * bi + jax.lax.broadcasted_iota(jnp.int32, (bi, bj), 0)
    col_g = jp * bj + jax.lax.broadcasted_iota(jnp.int32, (bi, bj), 1)

    @pl.when(j == 0)
    def _grab_diag():
        on_diag = row_g == col_g
        diag_ref[...] = jnp.sum(jnp.where(on_diag, sim, 0.0), axis=1)[:, None]

    ds = diag_ref[...]
    beats = (sim > ds) | ((sim == ds) & (col_g < row_g))
    beats = beats & (col_g != row_g)
    cnt_ref[...] += beats.astype(jnp.float32)

    @pl.when(j == nj - 1)
    def _finish_row_block():
        rank = jnp.sum(cnt_ref[...], axis=1)
        top1_ref[...] += jnp.sum((rank == 0.0).astype(jnp.float32))[None, None]
        top10_ref[...] += jnp.sum((rank < 10.0).astype(jnp.float32))[None, None]


def kernel(Z, Y):
    n, k = Z.shape
    bi = min(512, n)
    bj = min(512, n)
    ni = n // bi
    nj = n // bj

    def y_map(i, j):
        return (i, 0)

    def z_map(i, j):
        jd = (i * bi) // bj
        return (jax.lax.rem(jd + j, nj), 0)

    def out_map(i, j):
        return (0, 0)

    import functools
    body = functools.partial(_body, bi=bi, bj=bj, nj=nj)

    top1_sum, top10_sum = pl.pallas_call(
        body,
        grid=(ni, nj),
        in_specs=[
            pl.BlockSpec((bi, k), y_map),
            pl.BlockSpec((bj, k), z_map),
        ],
        out_specs=[
            pl.BlockSpec((1, 1), out_map),
            pl.BlockSpec((1, 1), out_map),
        ],
        out_shape=[
            jax.ShapeDtypeStruct((1, 1), jnp.float32),
            jax.ShapeDtypeStruct((1, 1), jnp.float32),
        ],
        scratch_shapes=[
            pltpu.VMEM((bi, bj), jnp.float32),
            pltpu.VMEM((bi, 1), jnp.float32),
            pltpu.VMEM((bi, 1), jnp.float32),
        ],
        compiler_params=pltpu.CompilerParams(
            dimension_semantics=("arbitrary", "arbitrary")),
    )(Y, Z)

    inv_n = jnp.float32(1.0 / n)
    return (top1_sum[0, 0] * inv_n, top10_sum[0, 0] * inv_n)


# speedup vs baseline: 4.4558x; 1.1931x over previous
import jax
import jax.numpy as jnp
from jax.experimental import pallas as pl
from jax.experimental.pallas import tpu as pltpu


def _body(y_ref, z_ref, top1_ref, top10_ref, cnt_ref, diag_ref, yn_ref,
          xn_ref, *, bi, bj, nj):
    i = pl.program_id(0)
    j = pl.program_id(1)

    y = y_ref[...]
    jd = (i * bi) // bj
    jp = jax.lax.rem(jd + j, nj)

    @pl.when(j == 0)
    def _init_row_block():
        cnt_ref[...] = jnp.zeros_like(cnt_ref)
        yn_ref[...] = jnp.sqrt(jnp.sum(y * y, axis=1))[:, None]

    @pl.when((i == 0) & (j == 0))
    def _init_outputs():
        top1_ref[...] = jnp.zeros_like(top1_ref)
        top10_ref[...] = jnp.zeros_like(top10_ref)

    @pl.when(i == 0)
    def _row_norms():
        z = z_ref[...]
        xn_ref[pl.ds(jp, 1), :] = jnp.sqrt(jnp.sum(z * z, axis=1))[None, :]

    dots = jax.lax.dot_general(
        y, z_ref[...], (((1,), (1,)), ((), ())),
        preferred_element_type=jnp.float32)
    xn = xn_ref[pl.ds(jp, 1), :]
    denom = jnp.maximum(yn_ref[...] * xn, 1e-8)
    sim = dots / denom

    row_g = i * bi + jax.lax.broadcasted_iota(jnp.int32, (bi, bj), 0)
    col_g = jp * bj + jax.lax.broadcasted_iota(jnp.int32, (bi, bj), 1)

    @pl.when(j == 0)
    def _grab_diag():
        on_diag = row_g == col_g
        diag_ref[...] = jnp.sum(jnp.where(on_diag, sim, 0.0), axis=1)[:, None]

    ds = diag_ref[...]
    beats = (sim > ds) | ((sim == ds) & (col_g < row_g))
    cnt_ref[...] += beats.astype(jnp.float32)

    @pl.when(j == nj - 1)
    def _finish_row_block():
        rank = jnp.sum(cnt_ref[...], axis=1)
        top1_ref[...] += jnp.sum((rank == 0.0).astype(jnp.float32))[None, None]
        top10_ref[...] += jnp.sum((rank < 10.0).astype(jnp.float32))[None, None]


def kernel(Z, Y):
    n, k = Z.shape
    bi = min(512, n)
    bj = min(512, n)
    ni = n // bi
    nj = n // bj

    def y_map(i, j):
        return (i, 0)

    def z_map(i, j):
        jd = (i * bi) // bj
        return (jax.lax.rem(jd + j, nj), 0)

    def out_map(i, j):
        return (0, 0)

    import functools
    body = functools.partial(_body, bi=bi, bj=bj, nj=nj)

    top1_sum, top10_sum = pl.pallas_call(
        body,
        grid=(ni, nj),
        in_specs=[
            pl.BlockSpec((bi, k), y_map),
            pl.BlockSpec((bj, k), z_map),
        ],
        out_specs=[
            pl.BlockSpec((1, 1), out_map),
            pl.BlockSpec((1, 1), out_map),
        ],
        out_shape=[
            jax.ShapeDtypeStruct((1, 1), jnp.float32),
            jax.ShapeDtypeStruct((1, 1), jnp.float32),
        ],
        scratch_shapes=[
            pltpu.VMEM((bi, bj), jnp.float32),
            pltpu.VMEM((bi, 1), jnp.float32),
            pltpu.VMEM((bi, 1), jnp.float32),
            pltpu.VMEM((nj, bj), jnp.float32),
        ],
        compiler_params=pltpu.CompilerParams(
            dimension_semantics=("arbitrary", "arbitrary")),
    )(Y, Z)

    inv_n = jnp.float32(1.0 / n)
    return (top1_sum[0, 0] * inv_n, top10_sum[0, 0] * inv_n)


# speedup vs baseline: 6.0739x; 1.3632x over previous
import jax
import jax.numpy as jnp
from jax.experimental import pallas as pl
from jax.experimental.pallas import tpu as pltpu


def _body(y_ref, z_ref, top1_ref, top10_ref, cnt_ref, diag_ref, yn_ref,
          xn_ref, *, bi, bj, nj):
    i = pl.program_id(0)
    j = pl.program_id(1)

    y = y_ref[...]
    jd = (i * bi) // bj
    jp = jax.lax.rem(jd + j, nj)

    @pl.when(j == 0)
    def _init_row_block():
        yn_ref[...] = jnp.sqrt(jnp.sum(y * y, axis=1))[:, None]

    @pl.when((i == 0) & (j == 0))
    def _init_outputs():
        top1_ref[...] = jnp.zeros_like(top1_ref)
        top10_ref[...] = jnp.zeros_like(top10_ref)

    @pl.when(i == 0)
    def _row_norms():
        z = z_ref[...]
        xn_ref[pl.ds(jp, 1), :] = jnp.sqrt(jnp.sum(z * z, axis=1))[None, :]

    dots = jax.lax.dot_general(
        y, z_ref[...], (((1,), (1,)), ((), ())),
        preferred_element_type=jnp.float32)
    xn = xn_ref[pl.ds(jp, 1), :]
    denom = jnp.maximum(yn_ref[...] * xn, 1e-8)
    sim = dots / denom

    @pl.when(j == 0)
    def _diag_tile():
        row_g = jax.lax.broadcasted_iota(jnp.int32, (bi, bj), 0)
        col_g = jax.lax.broadcasted_iota(jnp.int32, (bi, bj), 1)
        on_diag = row_g == col_g
        ds_vec = jnp.sum(jnp.where(on_diag, sim, 0.0), axis=1)[:, None]
        diag_ref[...] = ds_vec
        beats = (sim > ds_vec) | ((sim == ds_vec) & (col_g < row_g))
        cnt_ref[...] = beats.astype(jnp.float32)

    @pl.when(j != 0)
    def _off_diag_tile():
        ds = diag_ref[...]
        before_diag = jp < jd
        beats = (sim > ds) | ((sim == ds) & before_diag)
        cnt_ref[...] += beats.astype(jnp.float32)

    @pl.when(j == nj - 1)
    def _finish_row_block():
        rank = jnp.sum(cnt_ref[...], axis=1)
        top1_ref[...] += jnp.sum((rank == 0.0).astype(jnp.float32))[None, None]
        top10_ref[...] += jnp.sum((rank < 10.0).astype(jnp.float32))[None, None]


def kernel(Z, Y):
    n, k = Z.shape
    bi = min(1024, n)
    bj = min(1024, n)
    ni = n // bi
    nj = n // bj

    def y_map(i, j):
        return (i, 0)

    def z_map(i, j):
        jd = (i * bi) // bj
        return (jax.lax.rem(jd + j, nj), 0)

    def out_map(i, j):
        return (0, 0)

    import functools
    body = functools.partial(_body, bi=bi, bj=bj, nj=nj)

    top1_sum, top10_sum = pl.pallas_call(
        body,
        grid=(ni, nj),
        in_specs=[
            pl.BlockSpec((bi, k), y_map),
            pl.BlockSpec((bj, k), z_map),
        ],
        out_specs=[
            pl.BlockSpec((1, 1), out_map),
            pl.BlockSpec((1, 1), out_map),
        ],
        out_shape=[
            jax.ShapeDtypeStruct((1, 1), jnp.float32),
            jax.ShapeDtypeStruct((1, 1), jnp.float32),
        ],
        scratch_shapes=[
            pltpu.VMEM((bi, bj), jnp.float32),
            pltpu.VMEM((bi, 1), jnp.float32),
            pltpu.VMEM((bi, 1), jnp.float32),
            pltpu.VMEM((nj, bj), jnp.float32),
        ],
        compiler_params=pltpu.CompilerParams(
            dimension_semantics=("arbitrary", "arbitrary")),
    )(Y, Z)

    inv_n = jnp.float32(1.0 / n)
    return (top1_sum[0, 0] * inv_n, top10_sum[0, 0] * inv_n)
